# baseline (speedup 1.0000x reference)
import functools

import jax
import jax.numpy as jnp
from jax import lax
from jax.experimental import pallas as pl
from jax.experimental.pallas import tpu as pltpu
from jax.experimental.pallas import tpu_sc as plsc

_N = 2048
_W = 35
_C1 = 128
_C2 = 64
_E = 16384
_BS = 256
_EC = 8

_LANES = 16
_ROWS_PER_PASS = 32


def _adj_body(dst_hbm, src_hbm, ea_hbm, a_hbm, dstv, srcv, attrv, chunk):
    cid = lax.axis_index("c")
    sid = lax.axis_index("s")
    wid = sid * 2 + cid
    pltpu.sync_copy(dst_hbm, dstv)
    pltpu.sync_copy(src_hbm, srcv)
    pltpu.sync_copy(ea_hbm, attrv)
    zeros16 = jnp.zeros((_LANES,), jnp.float32)

    for p in range(2):
        base = wid * 64 + p * _ROWS_PER_PASS

        def zero_body(i, _, chunk=chunk):
            chunk[pl.ds(i * _LANES, _LANES)] = zeros16
            return 0

        lax.fori_loop(0, _ROWS_PER_PASS * (_N // _LANES), zero_body, 0)

        def edge_body(i, _, base=base, chunk=chunk):
            d = dstv[pl.ds(i * _LANES, _LANES)]
            inr = (d >= base) & (d < base + _ROWS_PER_PASS)
            s = srcv[pl.ds(i * _LANES, _LANES)]
            a = attrv[pl.ds(i * _LANES, _LANES)]
            flat = jnp.where(inr, (d - base) * _N + s, 0)
            plsc.addupdate_scatter(chunk, [flat], a, mask=inr)
            return 0

        lax.fori_loop(0, _E // _LANES, edge_body, 0)
        pltpu.sync_copy(
            chunk, a_hbm.at[pl.ds(base * _N, _ROWS_PER_PASS * _N)])


def _build_adj(edge_index, edge_attr):
    mesh = plsc.VectorSubcoreMesh(core_axis_name="c", subcore_axis_name="s")
    kern = functools.partial(
        pl.kernel,
        out_type=jax.ShapeDtypeStruct((_N * _N,), jnp.float32),
        mesh=mesh,
        scratch_types=[
            pltpu.VMEM((_E,), jnp.int32),
            pltpu.VMEM((_E,), jnp.int32),
            pltpu.VMEM((_E,), jnp.float32),
            pltpu.VMEM((_ROWS_PER_PASS * _N,), jnp.float32),
        ],
        compiler_params=pltpu.CompilerParams(
            needs_layout_passes=False, use_tc_tiling_on_sc=False),
    )(_adj_body)
    return kern(edge_index[1], edge_index[0], edge_attr).reshape(_N, _N)


def _split_w(xf, w, c):
    return jnp.stack([xf[:, i * c:(i + 1) * c] for i in range(w)], axis=1)


def _merge_w(x3):
    w = x3.shape[1]
    return jnp.concatenate([x3[:, i, :] for i in range(w)], axis=1)


def _tconv_mm(x3, w_ref, b_ref):
    bn, w, cin = x3.shape
    cout = w_ref.shape[2]
    zero = jnp.zeros((bn, 1, cin), jnp.float32)
    xp = jnp.concatenate([zero, x3, zero], axis=1)
    acc = jnp.broadcast_to(b_ref[...], (bn * w, cout))
    for k in range(3):
        acc = acc + jnp.dot(
            xp[:, k:k + w, :].reshape(bn * w, cin), w_ref[k],
            preferred_element_type=jnp.float32)
    return jnp.maximum(acc, 0.0).reshape(bn, w, cout)


def _tconv1_body(x_ref, w_ref, b_ref, o_ref):
    bn = x_ref.shape[0]
    x3 = _split_w(x_ref[...], _W, _C1)
    o_ref[...] = _merge_w(_tconv_mm(x3, w_ref, b_ref))


def _tconv1(xf, wk, b):
    n, f = xf.shape
    bn = 256
    return pl.pallas_call(
        _tconv1_body,
        grid=(n // bn,),
        in_specs=[
            pl.BlockSpec((bn, f), lambda i: (i, 0)),
            pl.BlockSpec((3, _C1, _C1), lambda i: (0, 0, 0)),
            pl.BlockSpec((1, _C1), lambda i: (0, 0)),
        ],
        out_specs=pl.BlockSpec((bn, f), lambda i: (i, 0)),
        out_shape=jax.ShapeDtypeStruct((n, f), jnp.float32),
    )(xf, wk, b.reshape(1, _C1))


def _spatial1_body(a_ref, x_ref, g_ref, bg_ref, w_ref, bt_ref,
                   o_ref, acc_ref, tsave_ref):
    i = pl.program_id(0)
    k = pl.program_id(1)
    nk = pl.num_programs(1)

    @pl.when(k == 0)
    def _():
        acc_ref[...] = jnp.zeros_like(acc_ref)

    acc_ref[...] += jnp.dot(a_ref[...], x_ref[...],
                            preferred_element_type=jnp.float32,
                            precision=lax.Precision.HIGHEST)

    @pl.when(k == i)
    def _():
        tsave_ref[...] = x_ref[...]

    @pl.when(k == nk - 1)
    def _():
        bn = acc_ref.shape[0]
        h3 = _split_w(tsave_ref[...] + acc_ref[...], _W, _C1)
        m = jnp.maximum(
            jnp.dot(h3.reshape(bn * _W, _C1), g_ref[...],
                    preferred_element_type=jnp.float32) + bg_ref[...], 0.0)
        m3 = m.reshape(bn, _W, _C1)
        o_ref[...] = _merge_w(_tconv_mm(m3, w_ref, bt_ref))


def _spatial1(adj, t1f, g, bg, wk, bt):
    n, f = t1f.shape
    bn = 256
    return pl.pallas_call(
        _spatial1_body,
        grid=(n // bn, n // bn),
        in_specs=[
            pl.BlockSpec((bn, bn), lambda i, k: (i, k)),
            pl.BlockSpec((bn, f), lambda i, k: (k, 0)),
            pl.BlockSpec((_C1, _C1), lambda i, k: (0, 0)),
            pl.BlockSpec((1, _C1), lambda i, k: (0, 0)),
            pl.BlockSpec((3, _C1, _C2), lambda i, k: (0, 0, 0)),
            pl.BlockSpec((1, _C2), lambda i, k: (0, 0)),
        ],
        out_specs=pl.BlockSpec((bn, _W * _C2), lambda i, k: (i, 0)),
        out_shape=jax.ShapeDtypeStruct((n, _W * _C2), jnp.float32),
        scratch_shapes=[pltpu.VMEM((bn, f), jnp.float32),
                        pltpu.VMEM((bn, f), jnp.float32)],
    )(adj, t1f, g, bg.reshape(1, _C1), wk, bt.reshape(1, _C2))


def _spatial2_body(a_ref, x_ref, g_ref, bg_ref, wc_ref, bc_ref,
                   wf_ref, bf_ref, o_ref, acc_ref, tsave_ref):
    i = pl.program_id(0)
    k = pl.program_id(1)
    nk = pl.num_programs(1)

    @pl.when(k == 0)
    def _():
        acc_ref[...] = jnp.zeros_like(acc_ref)

    acc_ref[...] += jnp.dot(a_ref[...], x_ref[...],
                            preferred_element_type=jnp.float32,
                            precision=lax.Precision.HIGHEST)

    @pl.when(k == i)
    def _():
        tsave_ref[...] = x_ref[...]

    @pl.when(k == nk - 1)
    def _():
        bn = acc_ref.shape[0]
        h3 = _split_w(tsave_ref[...] + acc_ref[...], _W, _C2)
        m = jnp.maximum(
            jnp.dot(h3.reshape(bn * _W, _C2), g_ref[...],
                    preferred_element_type=jnp.float32) + bg_ref[...], 0.0)
        m2f = _merge_w(m.reshape(bn, _W, _C2))
        c3 = jnp.dot(m2f, wc_ref[...],
                     preferred_element_type=jnp.float32) + bc_ref[...]
        c3r = c3.reshape(bn // _EC, _EC, 4 * _C2)
        flat = _merge_w(c3r)
        o_ref[...] = jnp.maximum(
            jnp.dot(flat, wf_ref[...], preferred_element_type=jnp.float32)
            + bf_ref[...], 0.0)


def _spatial2(adj, t2f, g, bg, wcbig, bctile, wf2, bf):
    n, f = t2f.shape
    bn = 256
    return pl.pallas_call(
        _spatial2_body,
        grid=(n // bn, n // bn),
        in_specs=[
            pl.BlockSpec((bn, bn), lambda i, k: (i, k)),
            pl.BlockSpec((bn, f), lambda i, k: (k, 0)),
            pl.BlockSpec((_C2, _C2), lambda i, k: (0, 0)),
            pl.BlockSpec((1, _C2), lambda i, k: (0, 0)),
            pl.BlockSpec((_W * _C2, 4 * _C2), lambda i, k: (0, 0)),
            pl.BlockSpec((1, 4 * _C2), lambda i, k: (0, 0)),
            pl.BlockSpec((_EC * 4 * _C2, 1), lambda i, k: (0, 0)),
            pl.BlockSpec((1, 1), lambda i, k: (0, 0)),
        ],
        out_specs=pl.BlockSpec((bn // _EC, 1), lambda i, k: (i, 0)),
        out_shape=jax.ShapeDtypeStruct((_BS, 1), jnp.float32),
        scratch_shapes=[pltpu.VMEM((bn, f), jnp.float32),
                        pltpu.VMEM((bn, f), jnp.float32)],
    )(adj, t2f, g, bg.reshape(1, _C2), wcbig, bctile,
      wf2, bf.reshape(1, 1))


def kernel(x, edge_index, edge_attr, batch, Wt1, bt1, Wg1, bg1,
           Wt2, bt2, Wg2, bg2, Wc, bc, Wf, bf):
    del batch
    w1 = jnp.transpose(Wt1, (2, 1, 0))
    w2 = jnp.transpose(Wt2, (2, 1, 0))
    g1 = Wg1.T
    g2 = Wg2.T
    wck = jnp.transpose(Wc, (2, 1, 0))
    cols = []
    for t in range(4):
        col = jnp.zeros((_W, _C2, _C2), jnp.float32)
        col = lax.dynamic_update_slice(col, wck, (t, 0, 0))
        cols.append(col.reshape(_W * _C2, _C2))
    wcbig = jnp.concatenate(cols, axis=1)
    bctile = jnp.tile(bc, (4,)).reshape(1, 4 * _C2)
    wf2 = Wf.reshape(_EC, _C2, 4).transpose(0, 2, 1).reshape(_EC * 4 * _C2, 1)

    adj = _build_adj(edge_index, edge_attr)
    t1 = _tconv1(x, w1, bt1)
    t2 = _spatial1(adj, t1, g1, bg1, w2, bt2)
    return _spatial2(adj, t2, g2, bg2, wcbig, bctile, wf2, bf)


# speedup vs baseline: 1.0294x; 1.0294x over previous
import functools

import jax
import jax.numpy as jnp
from jax import lax
from jax.experimental import pallas as pl
from jax.experimental.pallas import tpu as pltpu
from jax.experimental.pallas import tpu_sc as plsc

_N = 2048
_W = 35
_C1 = 128
_C2 = 64
_E = 16384
_BS = 256
_EC = 8

_LANES = 16
_ROWS_PER_PASS = 32


def _adj_body(dst_hbm, src_hbm, ea_hbm, a_hbm, dstv, srcv, attrv, chunk):
    cid = lax.axis_index("c")
    sid = lax.axis_index("s")
    wid = sid * 2 + cid
    pltpu.sync_copy(dst_hbm, dstv)
    pltpu.sync_copy(src_hbm, srcv)
    pltpu.sync_copy(ea_hbm, attrv)
    zeros16 = jnp.zeros((_LANES,), jnp.float32)

    for p in range(2):
        base = wid * 64 + p * _ROWS_PER_PASS

        def zero_body(i, _, chunk=chunk):
            chunk[pl.ds(i * _LANES, _LANES)] = zeros16
            return 0

        lax.fori_loop(0, _ROWS_PER_PASS * (_N // _LANES), zero_body, 0)

        def edge_body(i, _, base=base, chunk=chunk):
            d = dstv[pl.ds(i * _LANES, _LANES)]
            inr = (d >= base) & (d < base + _ROWS_PER_PASS)
            s = srcv[pl.ds(i * _LANES, _LANES)]
            a = attrv[pl.ds(i * _LANES, _LANES)]
            flat = jnp.where(inr, (d - base) * _N + s, 0)
            plsc.addupdate_scatter(chunk, [flat], a, mask=inr)
            return 0

        lax.fori_loop(0, _E // _LANES, edge_body, 0)
        pltpu.sync_copy(
            chunk, a_hbm.at[pl.ds(base * _N, _ROWS_PER_PASS * _N)])


def _build_adj(edge_index, edge_attr):
    mesh = plsc.VectorSubcoreMesh(core_axis_name="c", subcore_axis_name="s")
    kern = functools.partial(
        pl.kernel,
        out_type=jax.ShapeDtypeStruct((_N * _N,), jnp.float32),
        mesh=mesh,
        scratch_types=[
            pltpu.VMEM((_E,), jnp.int32),
            pltpu.VMEM((_E,), jnp.int32),
            pltpu.VMEM((_E,), jnp.float32),
            pltpu.VMEM((_ROWS_PER_PASS * _N,), jnp.float32),
        ],
        compiler_params=pltpu.CompilerParams(
            needs_layout_passes=False, use_tc_tiling_on_sc=False),
    )(_adj_body)
    return kern(edge_index[1], edge_index[0], edge_attr).reshape(_N, _N)


def _split_w(xf, w, c):
    return jnp.stack([xf[:, i * c:(i + 1) * c] for i in range(w)], axis=1)


def _merge_w(x3):
    w = x3.shape[1]
    return jnp.concatenate([x3[:, i, :] for i in range(w)], axis=1)


def _tconv_mm(x3, w_ref, b_ref):
    bn, w, cin = x3.shape
    cout = w_ref.shape[2]
    zero = jnp.zeros((bn, 1, cin), jnp.float32)
    xp = jnp.concatenate([zero, x3, zero], axis=1)
    acc = jnp.broadcast_to(b_ref[...], (bn * w, cout))
    for k in range(3):
        acc = acc + jnp.dot(
            xp[:, k:k + w, :].reshape(bn * w, cin), w_ref[k],
            preferred_element_type=jnp.float32)
    return jnp.maximum(acc, 0.0).reshape(bn, w, cout)


def _tconv1_body(x_ref, w_ref, b_ref, o_ref):
    bn = x_ref.shape[0]
    x3 = _split_w(x_ref[...], _W, _C1)
    o_ref[...] = _merge_w(_tconv_mm(x3, w_ref, b_ref))


def _tconv1(xf, wk, b):
    n, f = xf.shape
    bn = 256
    return pl.pallas_call(
        _tconv1_body,
        grid=(n // bn,),
        in_specs=[
            pl.BlockSpec((bn, f), lambda i: (i, 0)),
            pl.BlockSpec((3, _C1, _C1), lambda i: (0, 0, 0)),
            pl.BlockSpec((1, _C1), lambda i: (0, 0)),
        ],
        out_specs=pl.BlockSpec((bn, f), lambda i: (i, 0)),
        out_shape=jax.ShapeDtypeStruct((n, f), jnp.float32),
    )(xf, wk, b.reshape(1, _C1))


def _aggmm_body(a_ref, x_ref, o_ref):
    k = pl.program_id(1)

    @pl.when(k == 0)
    def _():
        o_ref[...] = jnp.zeros_like(o_ref)

    o_ref[...] += jnp.dot(a_ref[...], x_ref[...],
                          preferred_element_type=jnp.float32,
                          precision=lax.Precision.HIGHEST)


def _aggmm(adj, tf):
    n, f = tf.shape
    bn = 512
    nb = n // bn
    return pl.pallas_call(
        _aggmm_body,
        grid=(nb, nb),
        in_specs=[
            pl.BlockSpec((bn, bn), lambda i, k: (i, k)),
            pl.BlockSpec((bn, f), lambda i, k: (k, 0)),
        ],
        out_specs=pl.BlockSpec((bn, f), lambda i, k: (i, 0)),
        out_shape=jax.ShapeDtypeStruct((n, f), jnp.float32),
    )(adj, tf)


def _mixtconv_body(t_ref, agg_ref, g_ref, bg_ref, w_ref, bt_ref, o_ref):
    bn = t_ref.shape[0]
    h3 = _split_w(t_ref[...] + agg_ref[...], _W, _C1)
    h2 = h3.reshape(bn * _W, _C1)
    m = jnp.maximum(
        jnp.dot(h2, g_ref[...], preferred_element_type=jnp.float32)
        + bg_ref[...], 0.0)
    m3 = m.reshape(bn, _W, _C1)
    o_ref[...] = _merge_w(_tconv_mm(m3, w_ref, bt_ref))


def _mixtconv(t1f, agg1f, g, bg, wk, bt):
    n, f = t1f.shape
    bn = 256
    return pl.pallas_call(
        _mixtconv_body,
        grid=(n // bn,),
        in_specs=[
            pl.BlockSpec((bn, f), lambda i: (i, 0)),
            pl.BlockSpec((bn, f), lambda i: (i, 0)),
            pl.BlockSpec((_C1, _C1), lambda i: (0, 0)),
            pl.BlockSpec((1, _C1), lambda i: (0, 0)),
            pl.BlockSpec((3, _C1, _C2), lambda i: (0, 0, 0)),
            pl.BlockSpec((1, _C2), lambda i: (0, 0)),
        ],
        out_specs=pl.BlockSpec((bn, _W * _C2), lambda i: (i, 0)),
        out_shape=jax.ShapeDtypeStruct((n, _W * _C2), jnp.float32),
    )(t1f, agg1f, g, bg.reshape(1, _C1), wk, bt.reshape(1, _C2))


def _mixhead_body(t_ref, agg_ref, g_ref, bg_ref, wc_ref, bc_ref,
                  wf_ref, bf_ref, o_ref):
    bn = t_ref.shape[0]
    h3 = _split_w(t_ref[...] + agg_ref[...], _W, _C2)
    m = jnp.maximum(
        jnp.dot(h3.reshape(bn * _W, _C2), g_ref[...],
                preferred_element_type=jnp.float32) + bg_ref[...], 0.0)
    m2f = _merge_w(m.reshape(bn, _W, _C2))
    c3 = jnp.dot(m2f, wc_ref[...],
                 preferred_element_type=jnp.float32) + bc_ref[...]
    c3r = c3.reshape(bn // _EC, _EC, 4 * _C2)
    flat = _merge_w(c3r)
    o_ref[...] = jnp.maximum(
        jnp.dot(flat, wf_ref[...], preferred_element_type=jnp.float32)
        + bf_ref[...], 0.0)


def _mixhead(t2f, agg2f, g, bg, wcbig, bctile, wf2, bf):
    n, f = t2f.shape
    bn = 256
    return pl.pallas_call(
        _mixhead_body,
        grid=(n // bn,),
        in_specs=[
            pl.BlockSpec((bn, f), lambda i: (i, 0)),
            pl.BlockSpec((bn, f), lambda i: (i, 0)),
            pl.BlockSpec((_C2, _C2), lambda i: (0, 0)),
            pl.BlockSpec((1, _C2), lambda i: (0, 0)),
            pl.BlockSpec((_W * _C2, 4 * _C2), lambda i: (0, 0)),
            pl.BlockSpec((1, 4 * _C2), lambda i: (0, 0)),
            pl.BlockSpec((_EC * 4 * _C2, 1), lambda i: (0, 0)),
            pl.BlockSpec((1, 1), lambda i: (0, 0)),
        ],
        out_specs=pl.BlockSpec((bn // _EC, 1), lambda i: (i, 0)),
        out_shape=jax.ShapeDtypeStruct((_BS, 1), jnp.float32),
    )(t2f, agg2f, g, bg.reshape(1, _C2), wcbig, bctile,
      wf2, bf.reshape(1, 1))


def kernel(x, edge_index, edge_attr, batch, Wt1, bt1, Wg1, bg1,
           Wt2, bt2, Wg2, bg2, Wc, bc, Wf, bf):
    del batch
    w1 = jnp.transpose(Wt1, (2, 1, 0))
    w2 = jnp.transpose(Wt2, (2, 1, 0))
    g1 = Wg1.T
    g2 = Wg2.T
    wck = jnp.transpose(Wc, (2, 1, 0))
    cols = []
    for t in range(4):
        col = jnp.zeros((_W, _C2, _C2), jnp.float32)
        col = lax.dynamic_update_slice(col, wck, (t, 0, 0))
        cols.append(col.reshape(_W * _C2, _C2))
    wcbig = jnp.concatenate(cols, axis=1)
    bctile = jnp.tile(bc, (4,)).reshape(1, 4 * _C2)
    wf2 = Wf.reshape(_EC, _C2, 4).transpose(0, 2, 1).reshape(_EC * 4 * _C2, 1)

    adj = _build_adj(edge_index, edge_attr)
    t1 = _tconv1(x, w1, bt1)
    agg1 = _aggmm(adj, t1)
    t2 = _mixtconv(t1, agg1, g1, bg1, w2, bt2)
    agg2 = _aggmm(adj, t2)
    return _mixhead(t2, agg2, g2, bg2, wcbig, bctile, wf2, bf)
